# baseline (speedup 1.0000x reference)
import functools

import jax
import jax.numpy as jnp
from jax import lax
from jax.experimental import pallas as pl
from jax.experimental.pallas import tpu as pltpu
from jax.experimental.pallas import tpu_sc as plsc

N_NODES = 10000
N_PAD = 10240
NUM_GRAPHS = 64
N_EDGES = 320000
NC = 2
NS = 16
NW = NC * NS
EDGES_PER_W = 10240
E_PAD = NW * EDGES_PER_W
CHUNK = 512
SLAB = 128
SLABS = CHUNK // SLAB
CHUNKS_PER_W = EDGES_PER_W // CHUNK
ROWS_PER_TILE = N_PAD // NS


def _make_edge_kernel(D, CHUNK):
    H = D // 16
    SLABS = CHUNK // SLAB
    CHUNKS_PER_W = EDGES_PER_W // CHUNK
    mesh = plsc.VectorSubcoreMesh(
        core_axis_name="c", subcore_axis_name="s", num_cores=NC, num_subcores=NS
    )

    @functools.partial(
        pl.kernel,
        out_type=[
            jax.ShapeDtypeStruct((NC, N_PAD, D), jnp.float32),
            jax.ShapeDtypeStruct((NC, N_PAD), jnp.float32),
        ],
        mesh=mesh,
        compiler_params=pltpu.CompilerParams(
            needs_layout_passes=False, use_tc_tiling_on_sc=False),
        scratch_types=[
            pltpu.VMEM((2, SLABS, 3, SLAB), jnp.int32),
            pltpu.VMEM((2, CHUNK, D), jnp.float32),
            pltpu.VMEM((2, CHUNK, D), jnp.float32),
            pltpu.VMEM((2, CHUNK, D), jnp.float32),
            pltpu.VMEM((2, CHUNK), jnp.float32),
            pltpu.VMEM((2, D), jnp.float32),
            pltpu.VMEM((2, SLABS, SLAB), jnp.int32),
            pltpu.VMEM_SHARED((N_PAD, D), jnp.float32),
            pltpu.VMEM_SHARED((N_PAD,), jnp.float32),
            pltpu.SemaphoreType.DMA,
            pltpu.SemaphoreType.DMA,
            pltpu.SemaphoreType.DMA,
            pltpu.SemaphoreType.DMA,
        ],
    )
    def edge_kernel(
        xl_hbm, xr_hbm, pk_hbm, par_hbm, zrow_hbm, zden_hbm,
        acc_out, den_out,
        pk_v, xl_v, xr_v, out_v, ex_v, par_v, idx_sc,
        acc_sh, den_sh, sem_g0, sem_g1, sem_s0, sem_s1,
    ):
        gsems = (sem_g0, sem_g1)
        scsems = (sem_s0, sem_s1)
        cid = lax.axis_index("c")
        sid = lax.axis_index("s")
        wid = cid * NS + sid
        r0 = sid * ROWS_PER_TILE
        pltpu.sync_copy(zrow_hbm, acc_sh.at[pl.ds(r0, ROWS_PER_TILE)])
        pltpu.sync_copy(zden_hbm, den_sh.at[pl.ds(r0, ROWS_PER_TILE)])
        pltpu.sync_copy(par_hbm, par_v)
        plsc.subcore_barrier()

        lanes = lax.iota(jnp.int32, 16)
        we_h = [par_v[0, pl.ds(h * 16, 16)] for h in range(H)]
        att_h = [par_v[1, pl.ds(h * 16, 16)] for h in range(H)]
        lane_m = [lanes == t for t in range(16)]

        def stage(b, k):
            row0 = wid * (EDGES_PER_W // SLAB) + k * SLABS
            pltpu.sync_copy(pk_hbm.at[pl.ds(row0, SLABS)], pk_v.at[b])
            for j in range(SLABS):
                pltpu.async_copy(
                    xl_hbm.at[pk_v.at[b, j, 0]],
                    xl_v.at[b, pl.ds(j * SLAB, SLAB)], gsems[b])
                pltpu.async_copy(
                    xr_hbm.at[pk_v.at[b, j, 1]],
                    xr_v.at[b, pl.ds(j * SLAB, SLAB)], gsems[b])

        def drain(b):
            for j in range(SLABS):
                pltpu.make_async_copy(
                    xl_hbm.at[pk_v.at[b, j, 0]],
                    xl_v.at[b, pl.ds(j * SLAB, SLAB)], gsems[b]).wait()
                pltpu.make_async_copy(
                    xr_hbm.at[pk_v.at[b, j, 1]],
                    xr_v.at[b, pl.ds(j * SLAB, SLAB)], gsems[b]).wait()

        def wait_scatters(b):
            for j in range(SLABS):
                pltpu.make_async_copy(out_v.at[b, pl.ds(j * SLAB, SLAB)],
                                      acc_sh.at[idx_sc.at[b, j]],
                                      scsems[b]).wait()
                pltpu.make_async_copy(ex_v.at[b, pl.ds(j * SLAB, SLAB)],
                                      den_sh.at[idx_sc.at[b, j]],
                                      scsems[b]).wait()

        def compute_scatter(b):
            @plsc.parallel_loop(0, CHUNK // 16, 1, unroll=8)
            def group_body(g):
                e0 = pl.multiple_of(g * 16, 16)
                j = g // (SLAB // 16)
                off = pl.multiple_of((g % (SLAB // 16)) * 16, 16)
                ea_vec = plsc.bitcast(pk_v[b, j, 2, pl.ds(off, 16)], jnp.float32)
                logits = jnp.zeros((16,), jnp.float32)
                for t in range(16):
                    e = e0 + t
                    s = jnp.float32(0.0)
                    for h in range(H):
                        xlr = xl_v[b, e, pl.ds(h * 16, 16)]
                        xrr = xr_v[b, e, pl.ds(h * 16, 16)]
                        m = xlr + xrr + ea_vec[t] * we_h[h]
                        m = jnp.maximum(m, m * jnp.float32(0.2))
                        s = s + jnp.sum(m * att_h[h])
                    logits = jnp.where(lane_m[t], s, logits)
                ex = jnp.exp(logits)
                ex_v[b, pl.ds(e0, 16)] = ex
                for t in range(16):
                    e = e0 + t
                    for h in range(H):
                        out_v[b, e, pl.ds(h * 16, 16)] = (
                            ex[t] * xl_v[b, e, pl.ds(h * 16, 16)])

            for j in range(SLABS):
                for q in range(SLAB // 16):
                    idx_sc[b, j, pl.ds(q * 16, 16)] = (
                        pk_v[b, j, 1, pl.ds(q * 16, 16)])
            for j in range(SLABS):
                pltpu.async_copy(out_v.at[b, pl.ds(j * SLAB, SLAB)],
                                 acc_sh.at[idx_sc.at[b, j]],
                                 scsems[b], add=True)
                pltpu.async_copy(ex_v.at[b, pl.ds(j * SLAB, SLAB)],
                                 den_sh.at[idx_sc.at[b, j]],
                                 scsems[b], add=True)

        stage(0, 0)

        def pair_body(i, carry):
            k0 = i * 2
            stage(1, k0 + 1)
            drain(0)

            @pl.when(i > 0)
            def _():
                wait_scatters(0)

            compute_scatter(0)
            k2 = k0 + 2
            k2 = jnp.where(k2 >= CHUNKS_PER_W, 0, k2)
            stage(0, k2)
            drain(1)

            @pl.when(i > 0)
            def _():
                wait_scatters(1)

            compute_scatter(1)
            return carry

        lax.fori_loop(0, CHUNKS_PER_W // 2, pair_body, 0)
        drain(0)
        wait_scatters(0)
        wait_scatters(1)
        plsc.subcore_barrier()
        pltpu.sync_copy(acc_sh.at[pl.ds(r0, ROWS_PER_TILE)],
                        acc_out.at[cid, pl.ds(r0, ROWS_PER_TILE)])
        pltpu.sync_copy(den_sh.at[pl.ds(r0, ROWS_PER_TILE)],
                        den_out.at[cid, pl.ds(r0, ROWS_PER_TILE)])

    return edge_kernel


_edge16 = _make_edge_kernel(16, 1024)
_edge32 = _make_edge_kernel(32, 512)


def _proj_body(x_ref, ei_ref, ea_ref, wl_ref, wr_ref, xl_ref, xr_ref, pk_ref):
    x = x_ref[...]
    xl_ref[:N_NODES] = jnp.dot(x, wl_ref[...], preferred_element_type=jnp.float32)
    xr_ref[:N_NODES] = jnp.dot(x, wr_ref[...], preferred_element_type=jnp.float32)
    xl_ref[N_NODES:] = jnp.zeros((N_PAD - N_NODES, 16), jnp.float32)
    xr_ref[N_NODES:] = jnp.zeros((N_PAD - N_NODES, 16), jnp.float32)
    nrow = N_EDGES // SLAB
    prow = (E_PAD - N_EDGES) // SLAB
    pk_ref[:nrow, 0] = ei_ref[0]
    pk_ref[:nrow, 1] = ei_ref[1]
    r = lax.broadcasted_iota(jnp.int32, (prow, SLAB), 0)
    c = lax.broadcasted_iota(jnp.int32, (prow, SLAB), 1)
    pad_idx = N_NODES + (r * SLAB + c) % (N_PAD - N_NODES)
    pk_ref[nrow:, 0] = pad_idx
    pk_ref[nrow:, 1] = pad_idx
    pk_ref[:nrow, 2] = lax.bitcast_convert_type(ea_ref[...], jnp.int32)
    pk_ref[nrow:, 2] = jnp.zeros((prow, SLAB), jnp.int32)


def _mid_body(acc_ref, den_ref, b_ref, wl_ref, wr_ref, xl_ref, xr_ref):
    a = acc_ref[0] + acc_ref[1]
    d = den_ref[0] + den_ref[1]
    h = a / (d + jnp.float32(1e-16)) + b_ref[...][None, :]
    h = jnp.maximum(h, 0.0)
    xl_ref[...] = jnp.dot(h, wl_ref[...], preferred_element_type=jnp.float32)
    xr_ref[...] = jnp.dot(h, wr_ref[...], preferred_element_type=jnp.float32)


def _head_body(acc_ref, den_ref, b_ref, batch_ref,
               w4_ref, b4_ref, w5_ref, b5_ref, w6_ref, b6_ref, out_ref):
    a = acc_ref[0, :N_NODES] + acc_ref[1, :N_NODES]
    d = den_ref[0, :N_NODES] + den_ref[1, :N_NODES]
    h = a / (d + jnp.float32(1e-16)) + b_ref[...][None, :]
    h = jnp.maximum(h, 0.0)
    gid = lax.broadcasted_iota(jnp.int32, (N_NODES, NUM_GRAPHS), 1)
    p = jnp.where(batch_ref[...] == gid, 1.0, 0.0).astype(jnp.float32)
    sums = lax.dot_general(p, h, (((0,), (0,)), ((), ())),
                           preferred_element_type=jnp.float32)
    cnt = lax.dot_general(p, jnp.ones((N_NODES, 1), jnp.float32),
                          (((0,), (0,)), ((), ())),
                          preferred_element_type=jnp.float32)
    g = sums / jnp.maximum(cnt, 1.0)
    g = jnp.maximum(jnp.dot(g, w4_ref[...], preferred_element_type=jnp.float32)
                    + b4_ref[...][None, :], 0.0)
    g = jnp.maximum(jnp.dot(g, w5_ref[...], preferred_element_type=jnp.float32)
                    + b5_ref[...][None, :], 0.0)
    y = jnp.dot(g, w6_ref[...], preferred_element_type=jnp.float32) \
        + b6_ref[...][None, :]
    z = y - jnp.max(y, axis=1, keepdims=True)
    out_ref[...] = z - jnp.log(jnp.sum(jnp.exp(z), axis=1, keepdims=True))


def kernel(x, edge_index, edge_attr, batch,
           Wl1, Wr1, We1, att1, b1, Wl2, Wr2, We2, att2, b2,
           W4, b4, W5, b5, W6, b6):
    f32 = jnp.float32
    ei2d = edge_index.reshape(2, N_EDGES // SLAB, SLAB)
    batch2d = batch.reshape(N_NODES, 1)
    par1 = jnp.stack([We1[0], att1])
    par2 = jnp.stack([We2[0], att2])
    z16 = jnp.zeros((ROWS_PER_TILE, 16), f32)
    z32 = jnp.zeros((ROWS_PER_TILE, 32), f32)
    z1 = jnp.zeros((ROWS_PER_TILE,), f32)

    xl1, xr1, packed = pl.pallas_call(
        _proj_body,
        out_shape=[
            jax.ShapeDtypeStruct((N_PAD, 16), f32),
            jax.ShapeDtypeStruct((N_PAD, 16), f32),
            jax.ShapeDtypeStruct((E_PAD // SLAB, 3, SLAB), jnp.int32),
        ],
    )(x, ei2d, edge_attr[:, 0].reshape(N_EDGES // SLAB, SLAB), Wl1, Wr1)

    acc1, den1 = _edge16(xl1, xr1, packed, par1, z16, z1)

    xl2, xr2 = pl.pallas_call(
        _mid_body,
        out_shape=[jax.ShapeDtypeStruct((N_PAD, 32), f32)] * 2,
    )(acc1, den1.reshape(NC, N_PAD, 1), b1, Wl2, Wr2)

    acc2, den2 = _edge32(xl2, xr2, packed, par2, z32, z1)

    out = pl.pallas_call(
        _head_body,
        out_shape=jax.ShapeDtypeStruct((NUM_GRAPHS, 10), f32),
    )(acc2, den2.reshape(NC, N_PAD, 1), b2, batch2d, W4, b4, W5, b5, W6, b6)
    return out


# speedup vs baseline: 1.0639x; 1.0639x over previous
import functools

import jax
import jax.numpy as jnp
from jax import lax
from jax.experimental import pallas as pl
from jax.experimental.pallas import tpu as pltpu
from jax.experimental.pallas import tpu_sc as plsc

N_NODES = 10000
N_PAD = 10240
NUM_GRAPHS = 64
N_EDGES = 320000
NC = 2
NS = 16
NW = NC * NS
EDGES_PER_W = 10240
E_PAD = NW * EDGES_PER_W
CHUNK = 512
SLAB = 128
SLABS = CHUNK // SLAB
CHUNKS_PER_W = EDGES_PER_W // CHUNK
ROWS_PER_TILE = N_PAD // NS


def _make_edge_kernel(D, CHUNK):
    H = D // 16
    SLABS = CHUNK // SLAB
    CHUNKS_PER_W = EDGES_PER_W // CHUNK
    mesh = plsc.VectorSubcoreMesh(
        core_axis_name="c", subcore_axis_name="s", num_cores=NC, num_subcores=NS
    )

    @functools.partial(
        pl.kernel,
        out_type=[
            jax.ShapeDtypeStruct((NC, N_PAD, D), jnp.float32),
            jax.ShapeDtypeStruct((NC, N_PAD), jnp.float32),
        ],
        mesh=mesh,
        compiler_params=pltpu.CompilerParams(
            needs_layout_passes=False, use_tc_tiling_on_sc=False),
        scratch_types=[
            pltpu.VMEM((2, SLABS, 3, SLAB), jnp.int32),
            pltpu.VMEM((2, CHUNK, D), jnp.float32),
            pltpu.VMEM((2, CHUNK, D), jnp.float32),
            pltpu.VMEM((2, CHUNK, D), jnp.float32),
            pltpu.VMEM((2, CHUNK), jnp.float32),
            pltpu.VMEM((2, D), jnp.float32),
            pltpu.VMEM((2, SLABS, SLAB), jnp.int32),
            pltpu.VMEM_SHARED((N_PAD, D), jnp.float32),
            pltpu.VMEM_SHARED((N_PAD,), jnp.float32),
            pltpu.SemaphoreType.DMA,
            pltpu.SemaphoreType.DMA,
            pltpu.SemaphoreType.DMA,
            pltpu.SemaphoreType.DMA,
        ],
    )
    def edge_kernel(
        xl_hbm, xr_hbm, pk_hbm, par_hbm, zrow_hbm, zden_hbm,
        acc_out, den_out,
        pk_v, xl_v, xr_v, out_v, ex_v, par_v, idx_sc,
        acc_sh, den_sh, sem_g0, sem_g1, sem_s0, sem_s1,
    ):
        gsems = (sem_g0, sem_g1)
        scsems = (sem_s0, sem_s1)
        cid = lax.axis_index("c")
        sid = lax.axis_index("s")
        wid = cid * NS + sid
        r0 = sid * ROWS_PER_TILE
        pltpu.sync_copy(zrow_hbm, acc_sh.at[pl.ds(r0, ROWS_PER_TILE)])
        pltpu.sync_copy(zden_hbm, den_sh.at[pl.ds(r0, ROWS_PER_TILE)])
        pltpu.sync_copy(par_hbm, par_v)
        plsc.subcore_barrier()

        lanes = lax.iota(jnp.int32, 16)
        we_h = [par_v[0, pl.ds(h * 16, 16)] for h in range(H)]
        att_h = [par_v[1, pl.ds(h * 16, 16)] for h in range(H)]
        lane_m = [lanes == t for t in range(16)]

        def stage(b, k):
            row0 = wid * (EDGES_PER_W // SLAB) + k * SLABS
            pltpu.sync_copy(pk_hbm.at[pl.ds(row0, SLABS)], pk_v.at[b])
            for j in range(SLABS):
                pltpu.async_copy(
                    xl_hbm.at[pk_v.at[b, j, 0]],
                    xl_v.at[b, pl.ds(j * SLAB, SLAB)], gsems[b])
                pltpu.async_copy(
                    xr_hbm.at[pk_v.at[b, j, 1]],
                    xr_v.at[b, pl.ds(j * SLAB, SLAB)], gsems[b])

        def drain(b):
            for j in range(SLABS):
                pltpu.make_async_copy(
                    xl_hbm.at[pk_v.at[b, j, 0]],
                    xl_v.at[b, pl.ds(j * SLAB, SLAB)], gsems[b]).wait()
                pltpu.make_async_copy(
                    xr_hbm.at[pk_v.at[b, j, 1]],
                    xr_v.at[b, pl.ds(j * SLAB, SLAB)], gsems[b]).wait()

        def wait_scatters(b):
            for j in range(SLABS):
                pltpu.make_async_copy(out_v.at[b, pl.ds(j * SLAB, SLAB)],
                                      acc_sh.at[idx_sc.at[b, j]],
                                      scsems[b]).wait()
                pltpu.make_async_copy(ex_v.at[b, pl.ds(j * SLAB, SLAB)],
                                      den_sh.at[idx_sc.at[b, j]],
                                      scsems[b]).wait()

        def compute_scatter(b):
            @plsc.parallel_loop(0, CHUNK // 16, 1, unroll=4)
            def group_body(g):
                e0 = pl.multiple_of(g * 16, 16)
                j = g // (SLAB // 16)
                off = pl.multiple_of((g % (SLAB // 16)) * 16, 16)
                ea_vec = plsc.bitcast(pk_v[b, j, 2, pl.ds(off, 16)], jnp.float32)
                logits = jnp.zeros((16,), jnp.float32)
                for t in range(16):
                    e = e0 + t
                    s = jnp.float32(0.0)
                    for h in range(H):
                        xlr = xl_v[b, e, pl.ds(h * 16, 16)]
                        xrr = xr_v[b, e, pl.ds(h * 16, 16)]
                        m = xlr + xrr + ea_vec[t] * we_h[h]
                        m = jnp.maximum(m, m * jnp.float32(0.2))
                        s = s + jnp.sum(m * att_h[h])
                    logits = jnp.where(lane_m[t], s, logits)
                ex = jnp.exp(logits)
                ex_v[b, pl.ds(e0, 16)] = ex
                for t in range(16):
                    e = e0 + t
                    for h in range(H):
                        out_v[b, e, pl.ds(h * 16, 16)] = (
                            ex[t] * xl_v[b, e, pl.ds(h * 16, 16)])

            for j in range(SLABS):
                for q in range(SLAB // 16):
                    idx_sc[b, j, pl.ds(q * 16, 16)] = (
                        pk_v[b, j, 1, pl.ds(q * 16, 16)])
            for j in range(SLABS):
                pltpu.async_copy(out_v.at[b, pl.ds(j * SLAB, SLAB)],
                                 acc_sh.at[idx_sc.at[b, j]],
                                 scsems[b], add=True)
                pltpu.async_copy(ex_v.at[b, pl.ds(j * SLAB, SLAB)],
                                 den_sh.at[idx_sc.at[b, j]],
                                 scsems[b], add=True)

        stage(0, 0)

        def pair_body(i, carry):
            k0 = i * 2
            stage(1, k0 + 1)
            drain(0)

            @pl.when(i > 0)
            def _():
                wait_scatters(0)

            compute_scatter(0)
            k2 = k0 + 2
            k2 = jnp.where(k2 >= CHUNKS_PER_W, 0, k2)
            stage(0, k2)
            drain(1)

            @pl.when(i > 0)
            def _():
                wait_scatters(1)

            compute_scatter(1)
            return carry

        lax.fori_loop(0, CHUNKS_PER_W // 2, pair_body, 0)
        drain(0)
        wait_scatters(0)
        wait_scatters(1)
        plsc.subcore_barrier()
        pltpu.sync_copy(acc_sh.at[pl.ds(r0, ROWS_PER_TILE)],
                        acc_out.at[cid, pl.ds(r0, ROWS_PER_TILE)])
        pltpu.sync_copy(den_sh.at[pl.ds(r0, ROWS_PER_TILE)],
                        den_out.at[cid, pl.ds(r0, ROWS_PER_TILE)])

    return edge_kernel


_edge16 = _make_edge_kernel(16, 1024)
_edge32 = _make_edge_kernel(32, 512)


def _proj_body(x_ref, ei_ref, ea_ref, wl_ref, wr_ref, xl_ref, xr_ref, pk_ref):
    x = x_ref[...]
    xl_ref[:N_NODES] = jnp.dot(x, wl_ref[...], preferred_element_type=jnp.float32)
    xr_ref[:N_NODES] = jnp.dot(x, wr_ref[...], preferred_element_type=jnp.float32)
    xl_ref[N_NODES:] = jnp.zeros((N_PAD - N_NODES, 16), jnp.float32)
    xr_ref[N_NODES:] = jnp.zeros((N_PAD - N_NODES, 16), jnp.float32)
    nrow = N_EDGES // SLAB
    prow = (E_PAD - N_EDGES) // SLAB
    pk_ref[:nrow, 0] = ei_ref[0]
    pk_ref[:nrow, 1] = ei_ref[1]
    r = lax.broadcasted_iota(jnp.int32, (prow, SLAB), 0)
    c = lax.broadcasted_iota(jnp.int32, (prow, SLAB), 1)
    pad_idx = N_NODES + (r * SLAB + c) % (N_PAD - N_NODES)
    pk_ref[nrow:, 0] = pad_idx
    pk_ref[nrow:, 1] = pad_idx
    pk_ref[:nrow, 2] = lax.bitcast_convert_type(ea_ref[...], jnp.int32)
    pk_ref[nrow:, 2] = jnp.zeros((prow, SLAB), jnp.int32)


def _mid_body(acc_ref, den_ref, b_ref, wl_ref, wr_ref, xl_ref, xr_ref):
    a = acc_ref[0] + acc_ref[1]
    d = den_ref[0] + den_ref[1]
    h = a / (d + jnp.float32(1e-16)) + b_ref[...][None, :]
    h = jnp.maximum(h, 0.0)
    xl_ref[...] = jnp.dot(h, wl_ref[...], preferred_element_type=jnp.float32)
    xr_ref[...] = jnp.dot(h, wr_ref[...], preferred_element_type=jnp.float32)


def _head_body(acc_ref, den_ref, b_ref, batch_ref,
               w4_ref, b4_ref, w5_ref, b5_ref, w6_ref, b6_ref, out_ref):
    a = acc_ref[0, :N_NODES] + acc_ref[1, :N_NODES]
    d = den_ref[0, :N_NODES] + den_ref[1, :N_NODES]
    h = a / (d + jnp.float32(1e-16)) + b_ref[...][None, :]
    h = jnp.maximum(h, 0.0)
    gid = lax.broadcasted_iota(jnp.int32, (N_NODES, NUM_GRAPHS), 1)
    p = jnp.where(batch_ref[...] == gid, 1.0, 0.0).astype(jnp.float32)
    sums = lax.dot_general(p, h, (((0,), (0,)), ((), ())),
                           preferred_element_type=jnp.float32)
    cnt = lax.dot_general(p, jnp.ones((N_NODES, 1), jnp.float32),
                          (((0,), (0,)), ((), ())),
                          preferred_element_type=jnp.float32)
    g = sums / jnp.maximum(cnt, 1.0)
    g = jnp.maximum(jnp.dot(g, w4_ref[...], preferred_element_type=jnp.float32)
                    + b4_ref[...][None, :], 0.0)
    g = jnp.maximum(jnp.dot(g, w5_ref[...], preferred_element_type=jnp.float32)
                    + b5_ref[...][None, :], 0.0)
    y = jnp.dot(g, w6_ref[...], preferred_element_type=jnp.float32) \
        + b6_ref[...][None, :]
    z = y - jnp.max(y, axis=1, keepdims=True)
    out_ref[...] = z - jnp.log(jnp.sum(jnp.exp(z), axis=1, keepdims=True))


def kernel(x, edge_index, edge_attr, batch,
           Wl1, Wr1, We1, att1, b1, Wl2, Wr2, We2, att2, b2,
           W4, b4, W5, b5, W6, b6):
    f32 = jnp.float32
    ei2d = edge_index.reshape(2, N_EDGES // SLAB, SLAB)
    batch2d = batch.reshape(N_NODES, 1)
    par1 = jnp.stack([We1[0], att1])
    par2 = jnp.stack([We2[0], att2])
    z16 = jnp.zeros((ROWS_PER_TILE, 16), f32)
    z32 = jnp.zeros((ROWS_PER_TILE, 32), f32)
    z1 = jnp.zeros((ROWS_PER_TILE,), f32)

    xl1, xr1, packed = pl.pallas_call(
        _proj_body,
        out_shape=[
            jax.ShapeDtypeStruct((N_PAD, 16), f32),
            jax.ShapeDtypeStruct((N_PAD, 16), f32),
            jax.ShapeDtypeStruct((E_PAD // SLAB, 3, SLAB), jnp.int32),
        ],
    )(x, ei2d, edge_attr[:, 0].reshape(N_EDGES // SLAB, SLAB), Wl1, Wr1)

    acc1, den1 = _edge16(xl1, xr1, packed, par1, z16, z1)

    xl2, xr2 = pl.pallas_call(
        _mid_body,
        out_shape=[jax.ShapeDtypeStruct((N_PAD, 32), f32)] * 2,
    )(acc1, den1.reshape(NC, N_PAD, 1), b1, Wl2, Wr2)

    acc2, den2 = _edge32(xl2, xr2, packed, par2, z32, z1)

    out = pl.pallas_call(
        _head_body,
        out_shape=jax.ShapeDtypeStruct((NUM_GRAPHS, 10), f32),
    )(acc2, den2.reshape(NC, N_PAD, 1), b2, batch2d, W4, b4, W5, b5, W6, b6)
    return out
